# Initial kernel scaffold; baseline (speedup 1.0000x reference)
import jax
import jax.numpy as jnp
from jax.experimental import pallas as pl


def kernel(x, cond, params, batch_ids):
    raise NotImplementedError("write your pallas kernel here")


# speedup vs baseline: 67.9101x; 67.9101x over previous
import jax
import jax.numpy as jnp
from jax.experimental import pallas as pl
from jax.experimental.pallas import tpu as pltpu

B, P, NF, NC = 128, 512, 3, 6
K = 8
GNN_DIM, HEADS = 5, 3


def _leaky(v):
    return jnp.where(v > 0, v, 0.2 * v)


def _mm(a, b):
    return jnp.dot(a, b, preferred_element_type=jnp.float32)


def _outer_row(ones_col, col):
    return jax.lax.dot_general(
        ones_col, col, (((1,), (1,)), ((), ())),
        preferred_element_type=jnp.float32)


def _graph_kernel(x_ref, cond_ref,
                  cW0, cb0, cW1, cb1, cW2, cb2,
                  rW0, rb0, rW1, rb1,
                  lW0, lb0, lW1, lb1,
                  g1W0, g1b0, g1W1, g1b1,
                  gWl, gWr, gatt, gb,
                  g2W0, g2b0, g2W1, g2b1,
                  fW0, fb0, fW1, fb1,
                  out_ref):
    xb = x_ref[...]
    cond = cond_ref[...].reshape(1, NC)

    f = xb
    for Wt, b in ((cW0, cb0), (cW1, cb1), (cW2, cb2)):
        f = _leaky(_mm(f, Wt[...]) + b[...])
    feat = jnp.max(f, axis=0, keepdims=True)

    r = _leaky(_mm(feat, rW0[...]) + rb0[...])
    r = _mm(r, rW1[...]) + rb1[...]

    h = jnp.concatenate(
        [xb,
         jnp.broadcast_to(cond, (P, cond.shape[1])),
         jnp.broadcast_to(r, (P, r.shape[1]))], axis=1)
    xg = _leaky(_mm(h, lW0[...]) + lb0[...])
    xg = _mm(xg, lW1[...]) + lb1[...]

    ones_col = jnp.ones((P, 1), jnp.float32)
    sq = jnp.sum(xg * xg, axis=1, keepdims=True)
    gram = jax.lax.dot_general(
        xg, xg, (((1,), (1,)), ((), ())),
        preferred_element_type=jnp.float32)
    d2 = sq + _outer_row(ones_col, sq) - 2.0 * gram
    iota_q = jax.lax.broadcasted_iota(jnp.int32, (P, P), 1)
    mask = jnp.zeros((P, P), jnp.float32)
    d2m = d2
    for _ in range(K):
        cur = jnp.min(d2m, axis=1, keepdims=True)
        idx = jnp.min(jnp.where(d2m == cur, iota_q, P), axis=1,
                      keepdims=True)
        sel = iota_q == idx
        mask = jnp.where(sel, 1.0, mask)
        d2m = jnp.where(sel, jnp.float32(jnp.inf), d2m)

    y = xg + _mm(mask, xg)
    y = _leaky(_mm(y, g1W0[...]) + g1b0[...])
    xg = _mm(y, g1W1[...]) + g1b1[...]

    xl = _mm(xg, gWl[...])
    xr = _mm(xg, gWr[...])
    att = gatt[...]
    heads = []
    bool_mask = mask > 0.0
    for hh in range(HEADS):
        e = jnp.zeros((P, P), jnp.float32)
        for dd in range(GNN_DIM):
            c = hh * GNN_DIM + dd
            z = xr[:, c:c + 1] + _outer_row(ones_col, xl[:, c:c + 1])
            e = e + att[hh:hh + 1, dd:dd + 1] * _leaky(z)
        e_masked = jnp.where(bool_mask, e, jnp.float32(-1e30))
        emax = jnp.max(e_masked, axis=1, keepdims=True)
        ee = jnp.where(bool_mask, jnp.exp(e - emax), 0.0)
        den = jnp.sum(ee, axis=1, keepdims=True)
        alpha = ee / (den + 1e-16)
        heads.append(_mm(alpha, xl[:, hh * GNN_DIM:(hh + 1) * GNN_DIM]))
    xg = jnp.concatenate(heads, axis=1) + gb[...]

    y = xg + _mm(mask, xg)
    y = _leaky(_mm(y, g2W0[...]) + g2b0[...])
    xg = _mm(y, g2W1[...]) + g2b1[...]

    gnn = jnp.sum(xg, axis=0, keepdims=True)
    fin = jnp.concatenate([r, cond, gnn], axis=1)
    o = _leaky(_mm(fin, fW0[...]) + fb0[...])
    o = _mm(o, fW1[...]) + fb1[...]
    out_ref[...] = o.reshape(1, 1, 1)


def _flatten_params(params):
    flat = []

    def lin(layers):
        for W, b in layers:
            flat.append(W.T)
            flat.append(b.reshape(1, -1))

    lin(params["conv"])
    lin(params["rgan"])
    lin(params["lin"])
    lin(params["gin1"])
    gp = params["gat"]
    flat += [gp["Wl"].T, gp["Wr"].T, gp["att"], gp["b"].reshape(1, -1)]
    lin(params["gin2"])
    lin(params["final"])
    return flat


def kernel(x, cond, params, batch_ids):
    del batch_ids
    flat = _flatten_params(params)

    weight_specs = [
        pl.BlockSpec(w.shape, lambda b, _r=w.ndim: (0,) * _r)
        for w in flat
    ]
    out = pl.pallas_call(
        _graph_kernel,
        grid=(B,),
        in_specs=[
            pl.BlockSpec((P, NF), lambda b: (b, 0)),
            pl.BlockSpec((1, 1, NC), lambda b: (b, 0, 0)),
            *weight_specs,
        ],
        out_specs=pl.BlockSpec((1, 1, 1), lambda b: (b, 0, 0)),
        out_shape=jax.ShapeDtypeStruct((B, 1, 1), jnp.float32),
        compiler_params=pltpu.CompilerParams(
            dimension_semantics=("arbitrary",),
        ),
    )(x, cond.reshape(B, 1, NC), *flat)
    return out.reshape(B)
